# baseline (speedup 1.0000x reference)
import functools

import jax
import jax.numpy as jnp
from jax import lax
from jax.experimental import pallas as pl
from jax.experimental.pallas import tpu as pltpu
from jax.experimental.pallas import tpu_sc as plsc

B = 16384
FIELD = 26
V = 100000
EMB = 4
HIST = 50
CH = 80000
SEQ_EMB = 64
D1 = 32
D2 = 32
FE = FIELD * EMB

NW = 32
BPW = B // NW
NB = 32
NCH = BPW // NB

BLK = 2048


def _prep_body(seq_ref, w1b_ref, t2_ref):
    t2_ref[...] = jnp.dot(
        seq_ref[...], w1b_ref[...], preferred_element_type=jnp.float32
    ) * (1.0 / HIST)


def _xif_body(xi_ref, xif_ref):
    f = lax.broadcasted_iota(jnp.int32, xi_ref.shape, 1)
    xif_ref[...] = xi_ref[...] + f * V


def _comb_body(so_ref, fo_ref, comb_ref):
    z = jnp.zeros((so_ref.shape[0], 3), jnp.float32)
    comb_ref[...] = jnp.concatenate([so_ref[...], fo_ref[...], z], axis=1)


def _sc_body(t2, xseq, xif, comb,
             seqp_o, sof_o,
             xseq_v, rows_v, xif_v, sfrow_v, seqp_v):
    c = lax.axis_index("c")
    s = lax.axis_index("s")
    wid = c * 16 + s
    base0 = wid * BPW

    def chunk(i, carry):
        b0 = base0 + i * NB
        pltpu.sync_copy(xseq.at[pl.ds(b0 * HIST, NB * HIST)], xseq_v)
        pltpu.sync_copy(xif.at[pl.ds(b0 * FIELD, NB * FIELD)], xif_v)
        pltpu.sync_copy(t2.at[xseq_v], rows_v)
        pltpu.sync_copy(comb.at[xif_v], sfrow_v)

        def bb(b, carry2):
            base = b * HIST
            acc0 = jnp.zeros((16,), jnp.float32)
            acc1 = jnp.zeros((16,), jnp.float32)
            for h in range(HIST):
                acc0 = acc0 + rows_v[base + h, 0:16]
                acc1 = acc1 + rows_v[base + h, 16:32]
            seqp_v[b, 0:16] = acc0
            seqp_v[b, 16:32] = acc1
            return carry2

        lax.fori_loop(0, NB, bb, 0)
        pltpu.sync_copy(seqp_v, seqp_o.at[pl.ds(b0, NB)])
        pltpu.sync_copy(sfrow_v, sof_o.at[pl.ds(b0 * FIELD, NB * FIELD)])
        return carry

    lax.fori_loop(0, NCH, chunk, 0)


def _mlp_body(sog_ref, fog_ref, seqp_ref, xv_ref, xv4_ref,
              w1a_ref, w2_ref, wc1_ref, wc2_ref, wc3_ref,
              b1_ref, b2_ref, s_ref, out_ref):
    so = sog_ref[...] * xv4_ref[...]
    r = lax.broadcasted_iota(jnp.int32, (FE, EMB), 0)
    cc = lax.broadcasted_iota(jnp.int32, (FE, EMB), 1)
    sel = jnp.where((r % EMB) == cc, 1.0, 0.0).astype(jnp.float32)
    sum_emb = jnp.dot(so, sel, preferred_element_type=jnp.float32)
    sq_emb = jnp.dot(so * so, sel, preferred_element_type=jnp.float32)
    fm2 = 0.5 * (sum_emb * sum_emb - sq_emb)
    h1 = jnp.maximum(
        jnp.dot(so, w1a_ref[...], preferred_element_type=jnp.float32)
        + seqp_ref[...] + b1_ref[...], 0.0)
    h2 = jnp.maximum(
        jnp.dot(h1, w2_ref[...], preferred_element_type=jnp.float32)
        + b2_ref[...], 0.0)
    fm1 = fog_ref[...] * xv_ref[...]
    out = (jnp.dot(fm1, wc1_ref[...], preferred_element_type=jnp.float32)
           + jnp.dot(fm2, wc2_ref[...], preferred_element_type=jnp.float32)
           + jnp.dot(h2, wc3_ref[...], preferred_element_type=jnp.float32)
           + s_ref[0, 0])
    out_ref[...] = jax.nn.sigmoid(out)


def _make_sc_kernel():
    mesh = plsc.VectorSubcoreMesh(core_axis_name="c", subcore_axis_name="s",
                                  num_cores=2, num_subcores=16)
    return functools.partial(
        pl.kernel,
        out_type=(
            jax.ShapeDtypeStruct((B, D1), jnp.float32),
            jax.ShapeDtypeStruct((B * FIELD, 8), jnp.float32),
        ),
        mesh=mesh,
        scratch_types=[
            pltpu.VMEM((NB * HIST,), jnp.int32),
            pltpu.VMEM((NB * HIST, D1), jnp.float32),
            pltpu.VMEM((NB * FIELD,), jnp.int32),
            pltpu.VMEM((NB * FIELD, 8), jnp.float32),
            pltpu.VMEM((NB, D1), jnp.float32),
        ],
        compiler_params=pltpu.CompilerParams(
            use_tc_tiling_on_sc=False, needs_layout_passes=False),
    )(_sc_body)


def kernel(Xi, Xp, Xv, X_seq, fo_tables, so_tables, seq_table,
           W1, b1, W2, b2, Wc, bc, bias):
    idx = Xi[:, :, 0]
    so2d = so_tables.reshape(FIELD * V, EMB)
    fo2d = fo_tables.reshape(FIELD * V, 1)
    W1a = W1[:FE]
    W1b = W1[FE:]

    RB = 4000
    t2 = pl.pallas_call(
        _prep_body,
        grid=(CH // RB,),
        in_specs=[
            pl.BlockSpec((RB, SEQ_EMB), lambda i: (i, 0)),
            pl.BlockSpec((SEQ_EMB, D1), lambda i: (0, 0)),
        ],
        out_specs=pl.BlockSpec((RB, D1), lambda i: (i, 0)),
        out_shape=jax.ShapeDtypeStruct((CH, D1), jnp.float32),
    )(seq_table, W1b)
    XB = 4096
    xif = pl.pallas_call(
        _xif_body,
        grid=(B // XB,),
        in_specs=[pl.BlockSpec((XB, FIELD), lambda i: (i, 0))],
        out_specs=pl.BlockSpec((XB, FIELD), lambda i: (i, 0)),
        out_shape=jax.ShapeDtypeStruct((B, FIELD), jnp.int32),
    )(idx)
    RS = 5000
    comb = pl.pallas_call(
        _comb_body,
        grid=(FIELD * V // RS,),
        in_specs=[
            pl.BlockSpec((RS, EMB), lambda i: (i, 0)),
            pl.BlockSpec((RS, 1), lambda i: (i, 0)),
        ],
        out_specs=pl.BlockSpec((RS, 8), lambda i: (i, 0)),
        out_shape=jax.ShapeDtypeStruct((FIELD * V, 8), jnp.float32),
    )(so2d, fo2d)

    seqp, sof = _make_sc_kernel()(
        t2, X_seq.reshape(B * HIST), xif.reshape(B * FIELD), comb)
    sof3 = sof.reshape(B, FIELD, 8)
    sog = sof3[:, :, :EMB]
    fog = sof3[:, :, EMB]

    xv4 = jnp.repeat(Xv, EMB, axis=1)
    sog2 = sog.reshape(B, FE)
    sc = (bc + bias).reshape(1, 1)
    grid = B // BLK
    out = pl.pallas_call(
        _mlp_body,
        grid=(grid,),
        in_specs=[
            pl.BlockSpec((BLK, FE), lambda i: (i, 0)),
            pl.BlockSpec((BLK, FIELD), lambda i: (i, 0)),
            pl.BlockSpec((BLK, D1), lambda i: (i, 0)),
            pl.BlockSpec((BLK, FIELD), lambda i: (i, 0)),
            pl.BlockSpec((BLK, FE), lambda i: (i, 0)),
            pl.BlockSpec((FE, D1), lambda i: (0, 0)),
            pl.BlockSpec((D1, D2), lambda i: (0, 0)),
            pl.BlockSpec((FIELD, 1), lambda i: (0, 0)),
            pl.BlockSpec((EMB, 1), lambda i: (0, 0)),
            pl.BlockSpec((D2, 1), lambda i: (0, 0)),
            pl.BlockSpec((1, D1), lambda i: (0, 0)),
            pl.BlockSpec((1, D2), lambda i: (0, 0)),
            pl.BlockSpec((1, 1), lambda i: (0, 0)),
        ],
        out_specs=pl.BlockSpec((BLK, 1), lambda i: (i, 0)),
        out_shape=jax.ShapeDtypeStruct((B, 1), jnp.float32),
    )(sog2, fog, seqp, Xv, xv4,
      W1a, W2, Wc[:FIELD], Wc[FIELD:FIELD + EMB], Wc[FIELD + EMB:],
      b1.reshape(1, D1), b2.reshape(1, D2), sc)
    return out[:, 0]


# speedup vs baseline: 1.0617x; 1.0617x over previous
import functools

import jax
import jax.numpy as jnp
from jax import lax
from jax.experimental import pallas as pl
from jax.experimental.pallas import tpu as pltpu
from jax.experimental.pallas import tpu_sc as plsc

B = 16384
FIELD = 26
V = 100000
EMB = 4
HIST = 50
CH = 80000
SEQ_EMB = 64
D1 = 32
D2 = 32
FE = FIELD * EMB

NW = 32
BPW = B // NW
NB = 32
NCH = BPW // NB

BLK = 2048


def _prep_body(seq_ref, w1b_ref, t2_ref):
    t2_ref[...] = jnp.dot(
        seq_ref[...], w1b_ref[...], preferred_element_type=jnp.float32
    ) * (1.0 / HIST)


def _xif_body(xi_ref, xif_ref):
    f = lax.broadcasted_iota(jnp.int32, xi_ref.shape, 1)
    xif_ref[...] = xi_ref[...] + f * V


def _comb_body(so_ref, fo_ref, comb_ref):
    z = jnp.zeros((so_ref.shape[0], 32 - EMB - 1), jnp.float32)
    comb_ref[...] = jnp.concatenate([so_ref[...], fo_ref[...], z], axis=1)


def _sc_body(t2, xseq, xif, comb,
             seqp_o, sog_o, fog_o,
             xseq_v, rows_v, xif_v, sfrow_v, seqp_v, sog_v, fog_v):
    c = lax.axis_index("c")
    s = lax.axis_index("s")
    wid = c * 16 + s
    base0 = wid * BPW

    def chunk(i, carry):
        b0 = base0 + i * NB
        pltpu.sync_copy(xseq.at[pl.ds(b0 * HIST, NB * HIST)], xseq_v)
        pltpu.sync_copy(xif.at[pl.ds(b0 * FIELD, NB * FIELD)], xif_v)
        pltpu.sync_copy(t2.at[xseq_v], rows_v)
        pltpu.sync_copy(comb.at[xif_v], sfrow_v)

        def bb(b, carry2):
            base = b * HIST
            acc0 = jnp.zeros((16,), jnp.float32)
            acc1 = jnp.zeros((16,), jnp.float32)
            for h in range(HIST):
                acc0 = acc0 + rows_v[base + h, 0:16]
                acc1 = acc1 + rows_v[base + h, 16:32]
            seqp_v[b, 0:16] = acc0
            seqp_v[b, 16:32] = acc1
            return carry2

        lax.fori_loop(0, NB, bb, 0)

        lane = lax.iota(jnp.int32, 16)

        def rp_so(j, carry2):
            p = j * 16 + lane
            b = p >> 7
            q = p & 127
            f = jnp.minimum(q >> 2, FIELD - 1)
            e = q & 3
            val = plsc.load_gather(sfrow_v, [b * FIELD + f, e])
            sog_v[j >> 3, pl.ds((j & 7) * 16, 16)] = val
            return carry2

        lax.fori_loop(0, NB * 128 // 16, rp_so, 0)

        def rp_fo(j, carry2):
            p = j * 16 + lane
            b = p >> 5
            f = jnp.minimum(p & 31, FIELD - 1)
            val = plsc.load_gather(sfrow_v, [b * FIELD + f, jnp.full((16,), EMB, jnp.int32)])
            fog_v[j >> 1, pl.ds((j & 1) * 16, 16)] = val
            return carry2

        lax.fori_loop(0, NB * 32 // 16, rp_fo, 0)

        pltpu.sync_copy(seqp_v, seqp_o.at[pl.ds(b0, NB)])
        pltpu.sync_copy(sog_v, sog_o.at[pl.ds(b0, NB)])
        pltpu.sync_copy(fog_v, fog_o.at[pl.ds(b0, NB)])
        return carry

    lax.fori_loop(0, NCH, chunk, 0)


def _mlp_body(sog_ref, fog_ref, seqp_ref, xv_ref, xv4_ref,
              w1a_ref, w2_ref, wc1_ref, wc2_ref, wc3_ref,
              b1_ref, b2_ref, s_ref, out_ref):
    so = sog_ref[...] * xv4_ref[...]
    r = lax.broadcasted_iota(jnp.int32, (128, EMB), 0)
    cc = lax.broadcasted_iota(jnp.int32, (128, EMB), 1)
    sel = jnp.where(((r % EMB) == cc) & (r < FE), 1.0, 0.0).astype(jnp.float32)
    sum_emb = jnp.dot(so, sel, preferred_element_type=jnp.float32)
    sq_emb = jnp.dot(so * so, sel, preferred_element_type=jnp.float32)
    fm2 = 0.5 * (sum_emb * sum_emb - sq_emb)
    h1 = jnp.maximum(
        jnp.dot(so, w1a_ref[...], preferred_element_type=jnp.float32)
        + seqp_ref[...] + b1_ref[...], 0.0)
    h2 = jnp.maximum(
        jnp.dot(h1, w2_ref[...], preferred_element_type=jnp.float32)
        + b2_ref[...], 0.0)
    fm1 = fog_ref[...] * xv_ref[...]
    out = (jnp.dot(fm1, wc1_ref[...], preferred_element_type=jnp.float32)
           + jnp.dot(fm2, wc2_ref[...], preferred_element_type=jnp.float32)
           + jnp.dot(h2, wc3_ref[...], preferred_element_type=jnp.float32)
           + s_ref[0, 0])
    out_ref[...] = jax.nn.sigmoid(out)


def _make_sc_kernel():
    mesh = plsc.VectorSubcoreMesh(core_axis_name="c", subcore_axis_name="s",
                                  num_cores=2, num_subcores=16)
    return functools.partial(
        pl.kernel,
        out_type=(
            jax.ShapeDtypeStruct((B, D1), jnp.float32),
            jax.ShapeDtypeStruct((B, 128), jnp.float32),
            jax.ShapeDtypeStruct((B, 32), jnp.float32),
        ),
        mesh=mesh,
        scratch_types=[
            pltpu.VMEM((NB * HIST,), jnp.int32),
            pltpu.VMEM((NB * HIST, D1), jnp.float32),
            pltpu.VMEM((NB * FIELD,), jnp.int32),
            pltpu.VMEM((NB * FIELD, 32), jnp.float32),
            pltpu.VMEM((NB, D1), jnp.float32),
            pltpu.VMEM((NB, 128), jnp.float32),
            pltpu.VMEM((NB, 32), jnp.float32),
        ],
        compiler_params=pltpu.CompilerParams(
            use_tc_tiling_on_sc=False, needs_layout_passes=False),
    )(_sc_body)


def kernel(Xi, Xp, Xv, X_seq, fo_tables, so_tables, seq_table,
           W1, b1, W2, b2, Wc, bc, bias):
    idx = Xi[:, :, 0]
    so2d = so_tables.reshape(FIELD * V, EMB)
    fo2d = fo_tables.reshape(FIELD * V, 1)
    W1a = W1[:FE]
    W1b = W1[FE:]

    RB = 4000
    t2 = pl.pallas_call(
        _prep_body,
        grid=(CH // RB,),
        in_specs=[
            pl.BlockSpec((RB, SEQ_EMB), lambda i: (i, 0)),
            pl.BlockSpec((SEQ_EMB, D1), lambda i: (0, 0)),
        ],
        out_specs=pl.BlockSpec((RB, D1), lambda i: (i, 0)),
        out_shape=jax.ShapeDtypeStruct((CH, D1), jnp.float32),
    )(seq_table, W1b)
    XB = 4096
    xif = pl.pallas_call(
        _xif_body,
        grid=(B // XB,),
        in_specs=[pl.BlockSpec((XB, FIELD), lambda i: (i, 0))],
        out_specs=pl.BlockSpec((XB, FIELD), lambda i: (i, 0)),
        out_shape=jax.ShapeDtypeStruct((B, FIELD), jnp.int32),
    )(idx)
    RS = 5000
    comb = pl.pallas_call(
        _comb_body,
        grid=(FIELD * V // RS,),
        in_specs=[
            pl.BlockSpec((RS, EMB), lambda i: (i, 0)),
            pl.BlockSpec((RS, 1), lambda i: (i, 0)),
        ],
        out_specs=pl.BlockSpec((RS, 32), lambda i: (i, 0)),
        out_shape=jax.ShapeDtypeStruct((FIELD * V, 32), jnp.float32),
    )(so2d, fo2d)

    seqp, sog, fog = _make_sc_kernel()(
        t2, X_seq.reshape(B * HIST), xif.reshape(B * FIELD), comb)

    xv4 = jnp.pad(jnp.repeat(Xv, EMB, axis=1), ((0, 0), (0, 128 - FE)))
    xv32 = jnp.pad(Xv, ((0, 0), (0, 32 - FIELD)))
    w1a_pad = jnp.pad(W1a, ((0, 128 - FE), (0, 0)))
    wc1_pad = jnp.pad(Wc[:FIELD], ((0, 32 - FIELD), (0, 0)))
    sc = (bc + bias).reshape(1, 1)
    grid = B // BLK
    out = pl.pallas_call(
        _mlp_body,
        grid=(grid,),
        in_specs=[
            pl.BlockSpec((BLK, 128), lambda i: (i, 0)),
            pl.BlockSpec((BLK, 32), lambda i: (i, 0)),
            pl.BlockSpec((BLK, D1), lambda i: (i, 0)),
            pl.BlockSpec((BLK, 32), lambda i: (i, 0)),
            pl.BlockSpec((BLK, 128), lambda i: (i, 0)),
            pl.BlockSpec((128, D1), lambda i: (0, 0)),
            pl.BlockSpec((D1, D2), lambda i: (0, 0)),
            pl.BlockSpec((32, 1), lambda i: (0, 0)),
            pl.BlockSpec((EMB, 1), lambda i: (0, 0)),
            pl.BlockSpec((D2, 1), lambda i: (0, 0)),
            pl.BlockSpec((1, D1), lambda i: (0, 0)),
            pl.BlockSpec((1, D2), lambda i: (0, 0)),
            pl.BlockSpec((1, 1), lambda i: (0, 0)),
        ],
        out_specs=pl.BlockSpec((BLK, 1), lambda i: (i, 0)),
        out_shape=jax.ShapeDtypeStruct((B, 1), jnp.float32),
    )(sog, fog, seqp, xv32, xv4,
      w1a_pad, W2, wc1_pad, Wc[FIELD:FIELD + EMB], Wc[FIELD + EMB:],
      b1.reshape(1, D1), b2.reshape(1, D2), sc)
    return out[:, 0]


# speedup vs baseline: 1.3998x; 1.3184x over previous
import functools

import jax
import jax.numpy as jnp
from jax import lax
from jax.experimental import pallas as pl
from jax.experimental.pallas import tpu as pltpu
from jax.experimental.pallas import tpu_sc as plsc

B = 16384
FIELD = 26
V = 100000
EMB = 4
HIST = 50
CH = 80000
SEQ_EMB = 64
D1 = 32
D2 = 32
FE = FIELD * EMB

NW = 32
BPW = B // NW
NB = 32
NCH = BPW // NB

BLK = 2048


def _prep_body(seq_ref, w1b_ref, t2_ref):
    t2_ref[...] = jnp.dot(
        seq_ref[...], w1b_ref[...], preferred_element_type=jnp.float32
    ) * (1.0 / HIST)


def _xif_body(xi_ref, xif_ref):
    f = lax.broadcasted_iota(jnp.int32, xi_ref.shape, 1)
    xif_ref[...] = xi_ref[...] + f * V


def _comb_body(so_ref, fo_ref, comb_ref):
    x = so_ref[0]
    y = fo_ref[0]
    r = lax.broadcasted_iota(jnp.int32, (EMB, EMB), 0)
    c = lax.broadcasted_iota(jnp.int32, (EMB, EMB), 1)
    i4 = jnp.where(r == c, 1.0, 0.0).astype(jnp.float32)
    sot = lax.dot_general(x, i4, (((0,), (0,)), ((), ())),
                          preferred_element_type=jnp.float32)
    fot = lax.dot_general(y, jnp.ones((1, 1), jnp.float32),
                          (((0,), (0,)), ((), ())),
                          preferred_element_type=jnp.float32)
    z = jnp.zeros((sot.shape[0], 32 - EMB - 1), jnp.float32)
    comb_ref[...] = jnp.concatenate([sot, fot, z], axis=1)


def _sc_body(t2, xseq, xif, sot, fot,
             seqp_o, sog_o, fog_o,
             xseq_v, rows_v, xif_v, idx4_v, sorow_v, forow_v,
             seqp_v, sog_v, fog_v):
    c = lax.axis_index("c")
    s = lax.axis_index("s")
    wid = c * 16 + s
    base0 = wid * BPW

    def chunk(i, carry):
        b0 = base0 + i * NB
        pltpu.sync_copy(xseq.at[pl.ds(b0 * HIST, NB * HIST)], xseq_v)
        pltpu.sync_copy(xif.at[pl.ds(b0 * FIELD, NB * FIELD)], xif_v)

        lane0 = lax.iota(jnp.int32, 16)

        def expand(j, carry2):
            g = j * 16
            ln = lane0 + g
            src = ln >> 2
            val = plsc.load_gather(xif_v, [src])
            idx4_v[pl.ds(g, 16)] = val * EMB + (ln & 3)
            return carry2

        lax.fori_loop(0, NB * FE // 16, expand, 0)

        pltpu.sync_copy(t2.at[xseq_v], rows_v)
        pltpu.sync_copy(sot.at[idx4_v], sorow_v)
        pltpu.sync_copy(fot.at[xif_v], forow_v)

        def bb(b, carry2):
            base = b * HIST
            acc0 = jnp.zeros((16,), jnp.float32)
            acc1 = jnp.zeros((16,), jnp.float32)
            for h in range(HIST):
                acc0 = acc0 + rows_v[base + h, 0:16]
                acc1 = acc1 + rows_v[base + h, 16:32]
            seqp_v[b, 0:16] = acc0
            seqp_v[b, 16:32] = acc1
            return carry2

        lax.fori_loop(0, NB, bb, 0)

        def rp_so(j, carry2):
            p = j * 16 + lane0
            b = p >> 7
            q = p & 127
            src = b * FE + jnp.minimum(q, FE - 1)
            val = plsc.load_gather(sorow_v, [src])
            sog_v[j >> 3, pl.ds((j & 7) * 16, 16)] = val
            return carry2

        lax.fori_loop(0, NB * 128 // 16, rp_so, 0)

        def rp_fo(j, carry2):
            p = j * 16 + lane0
            b = p >> 5
            src = b * FIELD + jnp.minimum(p & 31, FIELD - 1)
            val = plsc.load_gather(forow_v, [src])
            fog_v[j >> 1, pl.ds((j & 1) * 16, 16)] = val
            return carry2

        lax.fori_loop(0, NB * 32 // 16, rp_fo, 0)

        pltpu.sync_copy(seqp_v, seqp_o.at[pl.ds(b0, NB)])
        pltpu.sync_copy(sog_v, sog_o.at[pl.ds(b0, NB)])
        pltpu.sync_copy(fog_v, fog_o.at[pl.ds(b0, NB)])
        return carry

    lax.fori_loop(0, NCH, chunk, 0)


def _mlp_body(sog_ref, fog_ref, seqp_ref, xv_ref,
              w1a_ref, w2_ref, wc1_ref, wc2_ref, wc3_ref,
              b1_ref, b2_ref, s_ref, out_ref):
    rr = lax.broadcasted_iota(jnp.int32, (32, 128), 0)
    cc4 = lax.broadcasted_iota(jnp.int32, (32, 128), 1)
    rep = jnp.where(((cc4 >> 2) == rr) & (cc4 < FE), 1.0, 0.0).astype(jnp.float32)
    xv4 = jnp.dot(xv_ref[...], rep, preferred_element_type=jnp.float32)
    so = sog_ref[...] * xv4
    r = lax.broadcasted_iota(jnp.int32, (128, EMB), 0)
    cc = lax.broadcasted_iota(jnp.int32, (128, EMB), 1)
    sel = jnp.where(((r % EMB) == cc) & (r < FE), 1.0, 0.0).astype(jnp.float32)
    sum_emb = jnp.dot(so, sel, preferred_element_type=jnp.float32)
    sq_emb = jnp.dot(so * so, sel, preferred_element_type=jnp.float32)
    fm2 = 0.5 * (sum_emb * sum_emb - sq_emb)
    h1 = jnp.maximum(
        jnp.dot(so, w1a_ref[...], preferred_element_type=jnp.float32)
        + seqp_ref[...] + b1_ref[...], 0.0)
    h2 = jnp.maximum(
        jnp.dot(h1, w2_ref[...], preferred_element_type=jnp.float32)
        + b2_ref[...], 0.0)
    fm1 = fog_ref[...] * xv_ref[...]
    out = (jnp.dot(fm1, wc1_ref[...], preferred_element_type=jnp.float32)
           + jnp.dot(fm2, wc2_ref[...], preferred_element_type=jnp.float32)
           + jnp.dot(h2, wc3_ref[...], preferred_element_type=jnp.float32)
           + s_ref[0, 0])
    out_ref[...] = jax.nn.sigmoid(out)


def _make_sc_kernel():
    mesh = plsc.VectorSubcoreMesh(core_axis_name="c", subcore_axis_name="s",
                                  num_cores=2, num_subcores=16)
    return functools.partial(
        pl.kernel,
        out_type=(
            jax.ShapeDtypeStruct((B, D1), jnp.float32),
            jax.ShapeDtypeStruct((B, 128), jnp.float32),
            jax.ShapeDtypeStruct((B, 32), jnp.float32),
        ),
        mesh=mesh,
        scratch_types=[
            pltpu.VMEM((NB * HIST,), jnp.int32),
            pltpu.VMEM((NB * HIST, D1), jnp.float32),
            pltpu.VMEM((NB * FIELD,), jnp.int32),
            pltpu.VMEM((NB * FE,), jnp.int32),
            pltpu.VMEM((NB * FE,), jnp.float32),
            pltpu.VMEM((NB * FIELD,), jnp.float32),
            pltpu.VMEM((NB, D1), jnp.float32),
            pltpu.VMEM((NB, 128), jnp.float32),
            pltpu.VMEM((NB, 32), jnp.float32),
        ],
        compiler_params=pltpu.CompilerParams(
            use_tc_tiling_on_sc=False, needs_layout_passes=False),
    )(_sc_body)


def kernel(Xi, Xp, Xv, X_seq, fo_tables, so_tables, seq_table,
           W1, b1, W2, b2, Wc, bc, bias):
    idx = Xi[:, :, 0]
    so_flat = so_tables.reshape(FIELD * V * EMB)
    fo_flat = fo_tables.reshape(FIELD * V)
    W1a = W1[:FE]
    W1b = W1[FE:]

    RB = 4000
    t2 = pl.pallas_call(
        _prep_body,
        grid=(CH // RB,),
        in_specs=[
            pl.BlockSpec((RB, SEQ_EMB), lambda i: (i, 0)),
            pl.BlockSpec((SEQ_EMB, D1), lambda i: (0, 0)),
        ],
        out_specs=pl.BlockSpec((RB, D1), lambda i: (i, 0)),
        out_shape=jax.ShapeDtypeStruct((CH, D1), jnp.float32),
    )(seq_table, W1b)
    XB = 4096
    xif = pl.pallas_call(
        _xif_body,
        grid=(B // XB,),
        in_specs=[pl.BlockSpec((XB, FIELD), lambda i: (i, 0))],
        out_specs=pl.BlockSpec((XB, FIELD), lambda i: (i, 0)),
        out_shape=jax.ShapeDtypeStruct((B, FIELD), jnp.int32),
    )(idx)
    seqp, sog, fog = _make_sc_kernel()(
        t2, X_seq.reshape(B * HIST), xif.reshape(B * FIELD), so_flat, fo_flat)

    xv32 = jnp.pad(Xv, ((0, 0), (0, 32 - FIELD)))
    w1a_pad = jnp.pad(W1a, ((0, 128 - FE), (0, 0)))
    wc1_pad = jnp.pad(Wc[:FIELD], ((0, 32 - FIELD), (0, 0)))
    sc = (bc + bias).reshape(1, 1)
    grid = B // BLK
    out = pl.pallas_call(
        _mlp_body,
        grid=(grid,),
        in_specs=[
            pl.BlockSpec((BLK, 128), lambda i: (i, 0)),
            pl.BlockSpec((BLK, 32), lambda i: (i, 0)),
            pl.BlockSpec((BLK, D1), lambda i: (i, 0)),
            pl.BlockSpec((BLK, 32), lambda i: (i, 0)),
            pl.BlockSpec((128, D1), lambda i: (0, 0)),
            pl.BlockSpec((D1, D2), lambda i: (0, 0)),
            pl.BlockSpec((32, 1), lambda i: (0, 0)),
            pl.BlockSpec((EMB, 1), lambda i: (0, 0)),
            pl.BlockSpec((D2, 1), lambda i: (0, 0)),
            pl.BlockSpec((1, D1), lambda i: (0, 0)),
            pl.BlockSpec((1, D2), lambda i: (0, 0)),
            pl.BlockSpec((1, 1), lambda i: (0, 0)),
        ],
        out_specs=pl.BlockSpec((BLK, 1), lambda i: (i, 0)),
        out_shape=jax.ShapeDtypeStruct((B, 1), jnp.float32),
    )(sog, fog, seqp, xv32,
      w1a_pad, W2, wc1_pad, Wc[FIELD:FIELD + EMB], Wc[FIELD + EMB:],
      b1.reshape(1, D1), b2.reshape(1, D2), sc)
    return out[:, 0]


# speedup vs baseline: 1.4003x; 1.0003x over previous
import functools

import jax
import jax.numpy as jnp
from jax import lax
from jax.experimental import pallas as pl
from jax.experimental.pallas import tpu as pltpu
from jax.experimental.pallas import tpu_sc as plsc

B = 16384
FIELD = 26
V = 100000
EMB = 4
HIST = 50
CH = 80000
SEQ_EMB = 64
D1 = 32
D2 = 32
FE = FIELD * EMB

NW = 32
BPW = B // NW
NB = 32
NCH = BPW // NB

BLK = 2048


def _prep_body(seq_ref, w1b_ref, t2_ref):
    t2_ref[...] = jnp.dot(
        seq_ref[...], w1b_ref[...], preferred_element_type=jnp.float32
    ) * (1.0 / HIST)


def _xif_body(xi_ref, xif_ref):
    f = lax.broadcasted_iota(jnp.int32, xi_ref.shape, 1)
    xif_ref[...] = xi_ref[...] + f * V


def _sc_body(t2, xseq, xif, sot, fot,
             seqp_o, sog_o, fog_o,
             xseq_v, rows_v, xif_v, idx4_v, sorow_v, forow_v,
             seqp_v, sog_v, fog_v):
    c = lax.axis_index("c")
    s = lax.axis_index("s")
    wid = c * 16 + s
    base0 = wid * BPW

    def chunk(i, carry):
        b0 = base0 + i * NB
        pltpu.sync_copy(xseq.at[pl.ds(b0 * HIST, NB * HIST)], xseq_v)
        pltpu.sync_copy(xif.at[pl.ds(b0 * FIELD, NB * FIELD)], xif_v)

        lane0 = lax.iota(jnp.int32, 16)

        def expand(j, carry2):
            g = j * 16
            ln = lane0 + g
            src = ln >> 2
            val = plsc.load_gather(xif_v, [src])
            idx4_v[pl.ds(g, 16)] = val * EMB + (ln & 3)
            return carry2

        lax.fori_loop(0, NB * FE // 16, expand, 0)

        pltpu.sync_copy(t2.at[xseq_v], rows_v)
        pltpu.sync_copy(sot.at[idx4_v], sorow_v)
        pltpu.sync_copy(fot.at[xif_v], forow_v)

        def bb(b, carry2):
            base = b * HIST
            acc0 = jnp.zeros((16,), jnp.float32)
            acc1 = jnp.zeros((16,), jnp.float32)
            for h in range(HIST):
                acc0 = acc0 + rows_v[base + h, 0:16]
                acc1 = acc1 + rows_v[base + h, 16:32]
            seqp_v[b, 0:16] = acc0
            seqp_v[b, 16:32] = acc1
            return carry2

        lax.fori_loop(0, NB, bb, 0)

        def rp_so(j, carry2):
            p = j * 16 + lane0
            b = p >> 7
            q = p & 127
            src = b * FE + jnp.minimum(q, FE - 1)
            val = plsc.load_gather(sorow_v, [src])
            sog_v[j >> 3, pl.ds((j & 7) * 16, 16)] = val
            return carry2

        lax.fori_loop(0, NB * 128 // 16, rp_so, 0)

        def rp_fo(j, carry2):
            p = j * 16 + lane0
            b = p >> 5
            src = b * FIELD + jnp.minimum(p & 31, FIELD - 1)
            val = plsc.load_gather(forow_v, [src])
            fog_v[j >> 1, pl.ds((j & 1) * 16, 16)] = val
            return carry2

        lax.fori_loop(0, NB * 32 // 16, rp_fo, 0)

        pltpu.sync_copy(seqp_v, seqp_o.at[pl.ds(b0, NB)])
        pltpu.sync_copy(sog_v, sog_o.at[pl.ds(b0, NB)])
        pltpu.sync_copy(fog_v, fog_o.at[pl.ds(b0, NB)])
        return carry

    lax.fori_loop(0, NCH, chunk, 0)


def _mlp_body(sog_ref, fog_ref, seqp_ref, xv_ref,
              w1a_ref, w2_ref, wc1_ref, wc2_ref, wc3_ref,
              b1_ref, b2_ref, s_ref, out_ref):
    rr = lax.broadcasted_iota(jnp.int32, (32, 128), 0)
    cc4 = lax.broadcasted_iota(jnp.int32, (32, 128), 1)
    rep = jnp.where(((cc4 >> 2) == rr) & (cc4 < FE), 1.0, 0.0).astype(jnp.float32)
    xv4 = jnp.dot(xv_ref[...], rep, preferred_element_type=jnp.float32)
    so = sog_ref[...] * xv4
    r = lax.broadcasted_iota(jnp.int32, (128, EMB), 0)
    cc = lax.broadcasted_iota(jnp.int32, (128, EMB), 1)
    sel = jnp.where(((r % EMB) == cc) & (r < FE), 1.0, 0.0).astype(jnp.float32)
    sum_emb = jnp.dot(so, sel, preferred_element_type=jnp.float32)
    sq_emb = jnp.dot(so * so, sel, preferred_element_type=jnp.float32)
    fm2 = 0.5 * (sum_emb * sum_emb - sq_emb)
    h1 = jnp.maximum(
        jnp.dot(so, w1a_ref[...], preferred_element_type=jnp.float32)
        + seqp_ref[...] + b1_ref[...], 0.0)
    h2 = jnp.maximum(
        jnp.dot(h1, w2_ref[...], preferred_element_type=jnp.float32)
        + b2_ref[...], 0.0)
    fm1 = fog_ref[...] * xv_ref[...]
    out = (jnp.dot(fm1, wc1_ref[...], preferred_element_type=jnp.float32)
           + jnp.dot(fm2, wc2_ref[...], preferred_element_type=jnp.float32)
           + jnp.dot(h2, wc3_ref[...], preferred_element_type=jnp.float32)
           + s_ref[0, 0])
    out_ref[...] = jax.nn.sigmoid(out)


def _make_sc_kernel():
    mesh = plsc.VectorSubcoreMesh(core_axis_name="c", subcore_axis_name="s",
                                  num_cores=2, num_subcores=16)
    return functools.partial(
        pl.kernel,
        out_type=(
            jax.ShapeDtypeStruct((B, D1), jnp.float32),
            jax.ShapeDtypeStruct((B, 128), jnp.float32),
            jax.ShapeDtypeStruct((B, 32), jnp.float32),
        ),
        mesh=mesh,
        scratch_types=[
            pltpu.VMEM((NB * HIST,), jnp.int32),
            pltpu.VMEM((NB * HIST, D1), jnp.float32),
            pltpu.VMEM((NB * FIELD,), jnp.int32),
            pltpu.VMEM((NB * FE,), jnp.int32),
            pltpu.VMEM((NB * FE,), jnp.float32),
            pltpu.VMEM((NB * FIELD,), jnp.float32),
            pltpu.VMEM((NB, D1), jnp.float32),
            pltpu.VMEM((NB, 128), jnp.float32),
            pltpu.VMEM((NB, 32), jnp.float32),
        ],
        compiler_params=pltpu.CompilerParams(
            use_tc_tiling_on_sc=False, needs_layout_passes=False),
    )(_sc_body)


def kernel(Xi, Xp, Xv, X_seq, fo_tables, so_tables, seq_table,
           W1, b1, W2, b2, Wc, bc, bias):
    idx = Xi[:, :, 0]
    so_flat = so_tables.reshape(FIELD * V * EMB)
    fo_flat = fo_tables.reshape(FIELD * V)
    W1a = W1[:FE]
    W1b = W1[FE:]

    RB = 4000
    t2 = pl.pallas_call(
        _prep_body,
        grid=(CH // RB,),
        in_specs=[
            pl.BlockSpec((RB, SEQ_EMB), lambda i: (i, 0)),
            pl.BlockSpec((SEQ_EMB, D1), lambda i: (0, 0)),
        ],
        out_specs=pl.BlockSpec((RB, D1), lambda i: (i, 0)),
        out_shape=jax.ShapeDtypeStruct((CH, D1), jnp.float32),
    )(seq_table, W1b)
    XB = 4096
    xif = pl.pallas_call(
        _xif_body,
        grid=(B // XB,),
        in_specs=[pl.BlockSpec((XB, FIELD), lambda i: (i, 0))],
        out_specs=pl.BlockSpec((XB, FIELD), lambda i: (i, 0)),
        out_shape=jax.ShapeDtypeStruct((B, FIELD), jnp.int32),
    )(idx)
    seqp, sog, fog = _make_sc_kernel()(
        t2, X_seq.reshape(B * HIST), xif.reshape(B * FIELD), so_flat, fo_flat)

    xv32 = jnp.pad(Xv, ((0, 0), (0, 32 - FIELD)))
    w1a_pad = jnp.pad(W1a, ((0, 128 - FE), (0, 0)))
    wc1_pad = jnp.pad(Wc[:FIELD], ((0, 32 - FIELD), (0, 0)))
    sc = (bc + bias).reshape(1, 1)
    grid = B // BLK
    out = pl.pallas_call(
        _mlp_body,
        grid=(grid,),
        in_specs=[
            pl.BlockSpec((BLK, 128), lambda i: (i, 0)),
            pl.BlockSpec((BLK, 32), lambda i: (i, 0)),
            pl.BlockSpec((BLK, D1), lambda i: (i, 0)),
            pl.BlockSpec((BLK, 32), lambda i: (i, 0)),
            pl.BlockSpec((128, D1), lambda i: (0, 0)),
            pl.BlockSpec((D1, D2), lambda i: (0, 0)),
            pl.BlockSpec((32, 1), lambda i: (0, 0)),
            pl.BlockSpec((EMB, 1), lambda i: (0, 0)),
            pl.BlockSpec((D2, 1), lambda i: (0, 0)),
            pl.BlockSpec((1, D1), lambda i: (0, 0)),
            pl.BlockSpec((1, D2), lambda i: (0, 0)),
            pl.BlockSpec((1, 1), lambda i: (0, 0)),
        ],
        out_specs=pl.BlockSpec((BLK, 1), lambda i: (i, 0)),
        out_shape=jax.ShapeDtypeStruct((B, 1), jnp.float32),
    )(sog, fog, seqp, xv32,
      w1a_pad, W2, wc1_pad, Wc[FIELD:FIELD + EMB], Wc[FIELD + EMB:],
      b1.reshape(1, D1), b2.reshape(1, D2), sc)
    return out[:, 0]
